# baseline (speedup 1.0000x reference)
import functools

import jax
import jax.numpy as jnp
from jax.experimental import pallas as pl

B, N, C = 8, 20000, 80
BN = 2000


def _fused_kernel(logits_ref, regress_ref, pts4_ref, centerness_ref,
                  logits_out_ref, boxes_out_ref):
    l = logits_ref[0]
    c = centerness_ref[0]
    logits_out_ref[0] = jnp.sqrt(l * c)

    r = regress_ref[0]
    lane = jax.lax.broadcasted_iota(jnp.int32, r.shape, 1)
    sign = jnp.where(lane >= 2, 1.0, -1.0).astype(jnp.float32)
    boxes = pts4_ref[...] + r * sign
    boxes_out_ref[0] = jnp.clip(boxes, 0.0, 1.0)


def kernel(logits, regress, points, centerness):
    pts4 = jnp.concatenate([points, points], axis=1)

    grid = (B, N // BN)
    out = pl.pallas_call(
        _fused_kernel,
        grid=grid,
        in_specs=[
            pl.BlockSpec((1, BN, C), lambda b, i: (b, i, 0)),
            pl.BlockSpec((1, BN, 4), lambda b, i: (b, i, 0)),
            pl.BlockSpec((BN, 4), lambda b, i: (i, 0)),
            pl.BlockSpec((1, BN, 1), lambda b, i: (b, i, 0)),
        ],
        out_specs=[
            pl.BlockSpec((1, BN, C), lambda b, i: (b, i, 0)),
            pl.BlockSpec((1, BN, 4), lambda b, i: (b, i, 0)),
        ],
        out_shape=[
            jax.ShapeDtypeStruct((B, N, C), jnp.float32),
            jax.ShapeDtypeStruct((B, N, 4), jnp.float32),
        ],
    )(logits, regress, pts4, centerness)
    return (out[0], out[1])
